# baseline (speedup 1.0000x reference)
import functools
import jax
import jax.numpy as jnp
from jax import lax
from jax.experimental import pallas as pl
from jax.experimental.pallas import tpu as pltpu, tpu_sc as plsc

B, L, D = 4096, 200, 64
V_TAB = 1000

_info = plsc.get_sparse_core_info()
NC, NS, LN = _info.num_cores, _info.num_subcores, _info.num_lanes
NW = NC * NS
PER_W = (B * L) // NW
ROWS_W = B // NW
GRP = ROWS_W // LN


def _body(mask_hbm, table_hbm, out_hbm, mask_v, tab_v, st0, st1, sw0, sw1):
    wid = lax.axis_index("s") * NC + lax.axis_index("c")
    flat0 = wid * PER_W
    lane0 = wid * ROWS_W

    pltpu.sync_copy(table_hbm, tab_v)
    pltpu.sync_copy(mask_hbm.at[pl.ds(flat0, PER_W)], mask_v)

    iota_l = lax.iota(jnp.int32, LN) * L
    zeros = jnp.zeros((LN,), jnp.int32)

    def cum_body(l, accs):
        new = []
        for g in range(GRP):
            idx = iota_l + (l + g * (LN * L))
            x = plsc.load_gather(mask_v, [idx])
            a = accs[g] + x
            plsc.store_scatter(mask_v, [idx], a * x)
            new.append(a)
        return tuple(new)

    lax.fori_loop(0, L, cum_body, (zeros,) * GRP)

    def build(l, stage):
        for g in range(GRP):
            p = plsc.load_gather(mask_v, [iota_l + (l + g * (LN * L))])
            pf = jnp.minimum(p, V_TAB - 1) * D
            for j in range(D):
                v = plsc.load_gather(tab_v, [pf + j])
                stage[j, pl.ds(g * LN, LN)] = v

    def wstart(l, stage, sem):
        pltpu.async_copy(
            stage, out_hbm.at[l, :, pl.ds(lane0, ROWS_W)], sem)

    def wwait(l, stage, sem):
        pltpu.make_async_copy(
            stage, out_hbm.at[l, :, pl.ds(lane0, ROWS_W)], sem).wait()

    build(0, st0)
    wstart(0, st0, sw0)

    def pair_body(i, _):
        l0 = 2 * i

        @pl.when(i > 0)
        def _():
            wwait(l0 - 1, st1, sw1)

        build(l0 + 1, st1)
        wstart(l0 + 1, st1, sw1)
        wwait(l0, st0, sw0)

        @pl.when(l0 + 2 < L)
        def _():
            build(l0 + 2, st0)
            wstart(l0 + 2, st0, sw0)

        return 0

    lax.fori_loop(0, L // 2, pair_body, jnp.int32(0))
    wwait(L - 1, st1, sw1)


@functools.partial(jax.jit, donate_argnums=())
def _run(mask_flat, table_flat):
    kern = pl.kernel(
        _body,
        out_type=jax.ShapeDtypeStruct((L, D, B), jnp.float32),
        mesh=plsc.VectorSubcoreMesh(core_axis_name="c", subcore_axis_name="s"),
        scratch_types=[
            pltpu.VMEM((PER_W,), jnp.int32),
            pltpu.VMEM((V_TAB * D,), jnp.float32),
            pltpu.VMEM((D, ROWS_W), jnp.float32),
            pltpu.VMEM((D, ROWS_W), jnp.float32),
            pltpu.SemaphoreType.DMA,
            pltpu.SemaphoreType.DMA,
        ],
        compiler_params=pltpu.CompilerParams(needs_layout_passes=False),
    )
    return kern(mask_flat, table_flat)


def kernel(input, mask, table):
    del input
    out = _run(mask.reshape(-1).astype(jnp.int32), table.reshape(-1))
    return jnp.transpose(out, (2, 0, 1))


# speedup vs baseline: 1.6244x; 1.6244x over previous
import functools
import jax
import jax.numpy as jnp
from jax import lax
from jax.experimental import pallas as pl
from jax.experimental.pallas import tpu as pltpu, tpu_sc as plsc

B, L, D = 4096, 200, 64
DS = D + 1
V_TAB = 1000

_info = plsc.get_sparse_core_info()
NC, NS, LN = _info.num_cores, _info.num_subcores, _info.num_lanes
NW = NC * NS
PER_W = (B * L) // NW
ROWS_W = B // NW
GRP = ROWS_W // LN


def _body(mask_hbm, table_hbm, out_hbm, mask_v, tab_v, st0, st1, sw0, sw1):
    wid = lax.axis_index("s") * NC + lax.axis_index("c")
    flat0 = wid * PER_W
    lane0 = wid * ROWS_W

    pltpu.sync_copy(table_hbm, tab_v.at[pl.ds(0, V_TAB * D)])
    pltpu.sync_copy(mask_hbm.at[pl.ds(flat0, PER_W)], mask_v)

    def skew_body(i, _):
        p = (V_TAB - 1) - i
        vs = [tab_v[pl.ds(p * D + k * LN, LN)] for k in range(D // LN)]
        for k in range(D // LN):
            tab_v[pl.ds(p * DS + k * LN, LN)] = vs[k]
        return 0

    lax.fori_loop(0, V_TAB, skew_body, jnp.int32(0))

    iota_l = lax.iota(jnp.int32, LN) * L
    zeros = jnp.zeros((LN,), jnp.int32)

    def cum_body(l, accs):
        new = []
        for g in range(GRP):
            idx = iota_l + (l + g * (LN * L))
            x = plsc.load_gather(mask_v, [idx])
            a = accs[g] + x
            plsc.store_scatter(mask_v, [idx], a * x)
            new.append(a)
        return tuple(new)

    lax.fori_loop(0, L, cum_body, (zeros,) * GRP)

    def build(l, stage):
        for g in range(GRP):
            p = plsc.load_gather(mask_v, [iota_l + (l + g * (LN * L))])
            pf = jnp.minimum(p, V_TAB - 1) * DS
            for j in range(D):
                v = plsc.load_gather(tab_v, [pf + j])
                stage[j, pl.ds(g * LN, LN)] = v

    def wstart(l, stage, sem):
        pltpu.async_copy(
            stage, out_hbm.at[l, :, pl.ds(lane0, ROWS_W)], sem)

    def wwait(l, stage, sem):
        pltpu.make_async_copy(
            stage, out_hbm.at[l, :, pl.ds(lane0, ROWS_W)], sem).wait()

    build(0, st0)
    wstart(0, st0, sw0)

    def pair_body(i, _):
        l0 = 2 * i

        @pl.when(i > 0)
        def _():
            wwait(l0 - 1, st1, sw1)

        build(l0 + 1, st1)
        wstart(l0 + 1, st1, sw1)
        wwait(l0, st0, sw0)

        @pl.when(l0 + 2 < L)
        def _():
            build(l0 + 2, st0)
            wstart(l0 + 2, st0, sw0)

        return 0

    lax.fori_loop(0, L // 2, pair_body, jnp.int32(0))
    wwait(L - 1, st1, sw1)


@functools.partial(jax.jit, donate_argnums=())
def _run(mask_flat, table_flat):
    kern = pl.kernel(
        _body,
        out_type=jax.ShapeDtypeStruct((L, D, B), jnp.float32),
        mesh=plsc.VectorSubcoreMesh(core_axis_name="c", subcore_axis_name="s"),
        scratch_types=[
            pltpu.VMEM((PER_W,), jnp.int32),
            pltpu.VMEM((V_TAB * DS,), jnp.float32),
            pltpu.VMEM((D, ROWS_W), jnp.float32),
            pltpu.VMEM((D, ROWS_W), jnp.float32),
            pltpu.SemaphoreType.DMA,
            pltpu.SemaphoreType.DMA,
        ],
        compiler_params=pltpu.CompilerParams(
            needs_layout_passes=False, disable_bounds_checks=True
        ),
    )
    return kern(mask_flat, table_flat)


def kernel(input, mask, table):
    del input
    out = _run(mask.reshape(-1).astype(jnp.int32), table.reshape(-1))
    return jnp.transpose(out, (2, 0, 1))


# speedup vs baseline: 7.5124x; 4.6249x over previous
import functools
import jax
import jax.numpy as jnp
from jax import lax
from jax.experimental import pallas as pl
from jax.experimental.pallas import tpu as pltpu, tpu_sc as plsc

B, L, D = 4096, 200, 64
DS = D + 1
V_TAB = 1000

_info = plsc.get_sparse_core_info()
NC, NS, LN = _info.num_cores, _info.num_subcores, _info.num_lanes
NW = NC * NS
PER_W = (B * L) // NW
ROWS_W = B // NW
GRP = ROWS_W // LN


def _body(mask_hbm, table_hbm, out_hbm, mask_v, tab_v, st0, st1, sw0, sw1):
    wid = lax.axis_index("s") * NC + lax.axis_index("c")
    flat0 = wid * PER_W
    lane0 = wid * ROWS_W

    pltpu.sync_copy(table_hbm, tab_v.at[pl.ds(0, V_TAB * D)])
    pltpu.sync_copy(mask_hbm.at[pl.ds(flat0, PER_W)], mask_v)

    def skew_body(i, _):
        p = (V_TAB - 1) - i
        vs = [tab_v[pl.ds(p * D + k * LN, LN)] for k in range(D // LN)]
        for k in range(D // LN):
            tab_v[pl.ds(p * DS + k * LN, LN)] = vs[k]
        return 0

    lax.fori_loop(0, V_TAB, skew_body, jnp.int32(0))

    iota_l = lax.iota(jnp.int32, LN) * L
    zeros = jnp.zeros((LN,), jnp.int32)

    def cum_body(l, accs):
        new = []
        for g in range(GRP):
            idx = iota_l + (l + g * (LN * L))
            x = plsc.load_gather(mask_v, [idx])
            a = accs[g] + x
            plsc.store_scatter(mask_v, [idx], a * x)
            new.append(a)
        return tuple(new)

    lax.fori_loop(0, L, cum_body, (zeros,) * GRP)

    def build(l, stage):
        for g in range(GRP):
            p = plsc.load_gather(mask_v, [iota_l + (l + g * (LN * L))])
            pf = jnp.minimum(p, V_TAB - 1) * DS

            @plsc.parallel_loop(0, D, unroll=8)
            def _(j):
                v = plsc.load_gather(tab_v, [pf + j])
                stage[j, pl.ds(g * LN, LN)] = v

    def wstart(l, stage, sem):
        pltpu.async_copy(
            stage, out_hbm.at[l, :, pl.ds(lane0, ROWS_W)], sem)

    def wwait(l, stage, sem):
        pltpu.make_async_copy(
            stage, out_hbm.at[l, :, pl.ds(lane0, ROWS_W)], sem).wait()

    build(0, st0)
    wstart(0, st0, sw0)

    def pair_body(i, _):
        l0 = 2 * i

        @pl.when(i > 0)
        def _():
            wwait(l0 - 1, st1, sw1)

        build(l0 + 1, st1)
        wstart(l0 + 1, st1, sw1)
        wwait(l0, st0, sw0)

        @pl.when(l0 + 2 < L)
        def _():
            build(l0 + 2, st0)
            wstart(l0 + 2, st0, sw0)

        return 0

    lax.fori_loop(0, L // 2, pair_body, jnp.int32(0))
    wwait(L - 1, st1, sw1)


@functools.partial(jax.jit, donate_argnums=())
def _run(mask_flat, table_flat):
    kern = pl.kernel(
        _body,
        out_type=jax.ShapeDtypeStruct((L, D, B), jnp.float32),
        mesh=plsc.VectorSubcoreMesh(core_axis_name="c", subcore_axis_name="s"),
        scratch_types=[
            pltpu.VMEM((PER_W,), jnp.int32),
            pltpu.VMEM((V_TAB * DS,), jnp.float32),
            pltpu.VMEM((D, ROWS_W), jnp.float32),
            pltpu.VMEM((D, ROWS_W), jnp.float32),
            pltpu.SemaphoreType.DMA,
            pltpu.SemaphoreType.DMA,
        ],
        compiler_params=pltpu.CompilerParams(
            needs_layout_passes=False, disable_bounds_checks=True
        ),
    )
    return kern(mask_flat, table_flat)


def kernel(input, mask, table):
    del input
    out = _run(mask.reshape(-1).astype(jnp.int32), table.reshape(-1))
    return jnp.transpose(out, (2, 0, 1))


# speedup vs baseline: 7.9383x; 1.0567x over previous
import functools
import jax
import jax.numpy as jnp
from jax import lax
from jax.experimental import pallas as pl
from jax.experimental.pallas import tpu as pltpu, tpu_sc as plsc

B, L, D = 4096, 200, 64
DS = D + 1
V_TAB = 1000

_info = plsc.get_sparse_core_info()
NC, NS, LN = _info.num_cores, _info.num_subcores, _info.num_lanes
NW = NC * NS
PER_W = (B * L) // NW
ROWS_W = B // NW
GRP = ROWS_W // LN


def _body(mask_hbm, table_hbm, out_hbm, mask_v, tab_v, st0, st1, sw0, sw1):
    wid = lax.axis_index("s") * NC + lax.axis_index("c")
    flat0 = wid * PER_W
    lane0 = wid * ROWS_W

    pltpu.sync_copy(table_hbm, tab_v.at[pl.ds(0, V_TAB * D)])
    pltpu.sync_copy(mask_hbm.at[pl.ds(flat0, PER_W)], mask_v)

    def skew_body(i, _):
        p = (V_TAB - 1) - i
        vs = [tab_v[pl.ds(p * D + k * LN, LN)] for k in range(D // LN)]
        for k in range(D // LN):
            tab_v[pl.ds(p * DS + k * LN, LN)] = vs[k]
        return 0

    lax.fori_loop(0, V_TAB, skew_body, jnp.int32(0))

    iota_l = lax.iota(jnp.int32, LN) * L
    zeros = jnp.zeros((LN,), jnp.int32)

    @plsc.parallel_loop(0, L, unroll=4, carry=(zeros,) * GRP)
    def _(l, accs):
        new = []
        for g in range(GRP):
            idx = iota_l + (l + g * (LN * L))
            x = plsc.load_gather(mask_v, [idx])
            a = accs[g] + x
            plsc.store_scatter(mask_v, [idx], a * x)
            new.append(a)
        return tuple(new)

    def build(l, stage):
        for g in range(GRP):
            p = plsc.load_gather(mask_v, [iota_l + (l + g * (LN * L))])
            pf = jnp.minimum(p, V_TAB - 1) * DS

            @plsc.parallel_loop(0, D, unroll=8)
            def _(j):
                v = plsc.load_gather(tab_v, [pf + j])
                stage[j, pl.ds(g * LN, LN)] = v

    def wstart(l, stage, sem):
        pltpu.async_copy(
            stage, out_hbm.at[l, :, pl.ds(lane0, ROWS_W)], sem)

    def wwait(l, stage, sem):
        pltpu.make_async_copy(
            stage, out_hbm.at[l, :, pl.ds(lane0, ROWS_W)], sem).wait()

    build(0, st0)
    wstart(0, st0, sw0)

    def pair_body(i, _):
        l0 = 2 * i

        @pl.when(i > 0)
        def _():
            wwait(l0 - 1, st1, sw1)

        build(l0 + 1, st1)
        wstart(l0 + 1, st1, sw1)
        wwait(l0, st0, sw0)

        @pl.when(l0 + 2 < L)
        def _():
            build(l0 + 2, st0)
            wstart(l0 + 2, st0, sw0)

        return 0

    lax.fori_loop(0, L // 2, pair_body, jnp.int32(0))
    wwait(L - 1, st1, sw1)


@functools.partial(jax.jit, donate_argnums=())
def _run(mask_flat, table_flat):
    kern = pl.kernel(
        _body,
        out_type=jax.ShapeDtypeStruct((L, D, B), jnp.float32),
        mesh=plsc.VectorSubcoreMesh(core_axis_name="c", subcore_axis_name="s"),
        scratch_types=[
            pltpu.VMEM((PER_W,), jnp.int32),
            pltpu.VMEM((V_TAB * DS,), jnp.float32),
            pltpu.VMEM((D, ROWS_W), jnp.float32),
            pltpu.VMEM((D, ROWS_W), jnp.float32),
            pltpu.SemaphoreType.DMA,
            pltpu.SemaphoreType.DMA,
        ],
        compiler_params=pltpu.CompilerParams(
            needs_layout_passes=False, disable_bounds_checks=True
        ),
    )
    return kern(mask_flat, table_flat)


def kernel(input, mask, table):
    del input
    out = _run(mask.reshape(-1).astype(jnp.int32), table.reshape(-1))
    return jnp.transpose(out, (2, 0, 1))
